# baseline (speedup 1.0000x reference)
import jax
import jax.numpy as jnp
from jax.experimental import pallas as pl
from jax.experimental.pallas import tpu as pltpu

_N = 65536
_GRID = 2
_BLK = _N // _GRID


def _add_body(x_ref, i_ref, o_ref):
    o_ref[...] = x_ref[...] + i_ref[...].astype(jnp.float32)


def kernel(input_xyzs, query_xyz_index):
    x = input_xyzs.T
    i = query_xyz_index.T
    out = pl.pallas_call(
        _add_body,
        grid=(_GRID,),
        in_specs=[
            pl.BlockSpec((3, _BLK), lambda g: (0, g)),
            pl.BlockSpec((3, _BLK), lambda g: (0, g)),
        ],
        out_specs=pl.BlockSpec((3, _BLK), lambda g: (0, g)),
        out_shape=jax.ShapeDtypeStruct((3, _N), jnp.float32),
        compiler_params=pltpu.CompilerParams(
            dimension_semantics=("arbitrary",),
        ),
    )(x, i)
    return out.T


# speedup vs baseline: 1.0115x; 1.0115x over previous
import jax
import jax.numpy as jnp
from jax.experimental import pallas as pl
from jax.experimental.pallas import tpu as pltpu

_N = 65536
_GRID = 2
_BLK = _N // _GRID


def _add_body(x_ref, i_ref, o_ref):
    o_ref[...] = x_ref[...] + i_ref[...].astype(jnp.float32)


def kernel(input_xyzs, query_xyz_index):
    x = input_xyzs.T
    i = query_xyz_index.T
    out = pl.pallas_call(
        _add_body,
        grid=(_GRID,),
        in_specs=[
            pl.BlockSpec((3, _BLK), lambda g: (0, g)),
            pl.BlockSpec((3, _BLK), lambda g: (0, g)),
        ],
        out_specs=pl.BlockSpec((3, _BLK), lambda g: (0, g)),
        out_shape=jax.ShapeDtypeStruct((3, _N), jnp.float32),
        compiler_params=pltpu.CompilerParams(
            dimension_semantics=("parallel",),
        ),
    )(x, i)
    return out.T
